# baseline (speedup 1.0000x reference)
import functools
import math

import jax
import jax.numpy as jnp
from jax import lax
from jax.experimental import pallas as pl
from jax.experimental.pallas import tpu as pltpu
from jax.experimental.pallas import tpu_sc as plsc

B = 16384
D = 128
L = 16
NC = 2
NS = 16
NW = NC * NS
BPW = B // NW
CHUNK = 128
NCHUNK = BPW // CHUNK
VPR = D // L


def _make_sc_fn():
  mesh = plsc.VectorSubcoreMesh(core_axis_name="c", subcore_axis_name="s")

  @functools.partial(
      pl.kernel,
      mesh=mesh,
      out_type=jax.ShapeDtypeStruct((NW, L), jnp.float32),
      scratch_types=[
          pltpu.VMEM((NCHUNK, CHUNK), jnp.int32),
          pltpu.VMEM((CHUNK, D), jnp.float32),
          pltpu.VMEM((CHUNK, D), jnp.float32),
          pltpu.VMEM((BPW,), jnp.float32),
          pltpu.VMEM((L,), jnp.float32),
          pltpu.SemaphoreType.DMA,
      ],
  )
  def sc_fn(z_hbm, sldj_hbm, lab_hbm, means_hbm, out_hbm,
            idx_v, z_v, rows_v, sldj_v, part_v, sem):
    wid = lax.axis_index("s") * NC + lax.axis_index("c")
    pltpu.sync_copy(lab_hbm.at[wid], idx_v)
    pltpu.sync_copy(sldj_hbm.at[wid], sldj_v)

    zero = jnp.zeros((L,), jnp.float32)

    def chunk_body(c, accs):
      pltpu.async_copy(means_hbm.at[idx_v.at[c]], rows_v, sem).wait()
      pltpu.sync_copy(z_hbm.at[wid, c], z_v)

      def row_body(r, a):
        new = []
        for v in range(VPR):
          diff = z_v[r, pl.ds(v * L, L)] - rows_v[r, pl.ds(v * L, L)]
          new.append(a[v] + diff * diff)
        return tuple(new)

      return lax.fori_loop(0, CHUNK, row_body, accs)

    accs = lax.fori_loop(0, NCHUNK, chunk_body, (zero,) * VPR)
    sq = accs[0]
    for v in range(1, VPR):
      sq = sq + accs[v]

    def sldj_body(i, a):
      return a + sldj_v[pl.ds(i * L, L)]

    sacc = lax.fori_loop(0, BPW // L, sldj_body, zero)

    part_v[...] = 0.5 * sq - sacc
    pltpu.sync_copy(part_v, out_hbm.at[wid])

  return sc_fn


_sc_fn = _make_sc_fn()


def kernel(z, sldj, labels, means):
  zr = z.reshape(NW, NCHUNK, CHUNK, D)
  sldjr = sldj.reshape(NW, BPW)
  labr = labels.astype(jnp.int32).reshape(NW, NCHUNK, CHUNK)
  parts = _sc_fn(zr, sldjr, labr, means)
  const = 0.5 * D * math.log(2.0 * math.pi)
  return parts.sum() / B + const


# speedup vs baseline: 1.2260x; 1.2260x over previous
import functools
import math

import jax
import jax.numpy as jnp
from jax import lax
from jax.experimental import pallas as pl
from jax.experimental.pallas import tpu as pltpu
from jax.experimental.pallas import tpu_sc as plsc

B = 16384
D = 128
L = 16
NC = 2
NS = 16
NW = NC * NS
BPW = B // NW
CHUNK = 128
NCHUNK = BPW // CHUNK
VPR = D // L


def _make_sc_fn():
  mesh = plsc.VectorSubcoreMesh(core_axis_name="c", subcore_axis_name="s")

  @functools.partial(
      pl.kernel,
      mesh=mesh,
      out_type=jax.ShapeDtypeStruct((NW, L), jnp.float32),
      scratch_types=[
          pltpu.VMEM((NCHUNK, CHUNK), jnp.int32),
          pltpu.VMEM((CHUNK, D), jnp.float32),
          pltpu.VMEM((CHUNK, D), jnp.float32),
          pltpu.VMEM((CHUNK, D), jnp.float32),
          pltpu.VMEM((CHUNK, D), jnp.float32),
          pltpu.VMEM((BPW,), jnp.float32),
          pltpu.VMEM((L,), jnp.float32),
          pltpu.SemaphoreType.DMA,
          pltpu.SemaphoreType.DMA,
      ],
  )
  def sc_fn(z_hbm, sldj_hbm, lab_hbm, means_hbm, out_hbm,
            idx_v, z_v0, z_v1, rows_v0, rows_v1, sldj_v, part_v,
            gsem, zsem):
    wid = lax.axis_index("s") * NC + lax.axis_index("c")
    pltpu.sync_copy(lab_hbm.at[wid], idx_v)

    z_bufs = (z_v0, z_v1)
    row_bufs = (rows_v0, rows_v1)
    gcp = [None] * NCHUNK
    zcp = [None] * NCHUNK
    gcp[0] = pltpu.async_copy(means_hbm.at[idx_v.at[0]], rows_v0, gsem)
    zcp[0] = pltpu.async_copy(z_hbm.at[wid, 0], z_v0, zsem)

    pltpu.sync_copy(sldj_hbm.at[wid], sldj_v)

    zero = jnp.zeros((L,), jnp.float32)
    accs = (zero,) * VPR
    for c in range(NCHUNK):
      if c + 1 < NCHUNK:
        nb = (c + 1) % 2
        gcp[c + 1] = pltpu.async_copy(
            means_hbm.at[idx_v.at[c + 1]], row_bufs[nb], gsem)
        zcp[c + 1] = pltpu.async_copy(z_hbm.at[wid, c + 1], z_bufs[nb], zsem)
      gcp[c].wait()
      zcp[c].wait()
      z_v = z_bufs[c % 2]
      rows_v = row_bufs[c % 2]

      def row_body(r, a, z_v=z_v, rows_v=rows_v):
        new = []
        for v in range(VPR):
          diff = z_v[r, pl.ds(v * L, L)] - rows_v[r, pl.ds(v * L, L)]
          new.append(a[v] + diff * diff)
        return tuple(new)

      accs = lax.fori_loop(0, CHUNK, row_body, accs)
    sq = accs[0]
    for v in range(1, VPR):
      sq = sq + accs[v]

    def sldj_body(i, a):
      return a + sldj_v[pl.ds(i * L, L)]

    sacc = lax.fori_loop(0, BPW // L, sldj_body, zero)

    part_v[...] = 0.5 * sq - sacc
    pltpu.sync_copy(part_v, out_hbm.at[wid])

  return sc_fn


_sc_fn = _make_sc_fn()


def kernel(z, sldj, labels, means):
  zr = z.reshape(NW, NCHUNK, CHUNK, D)
  sldjr = sldj.reshape(NW, BPW)
  labr = labels.astype(jnp.int32).reshape(NW, NCHUNK, CHUNK)
  parts = _sc_fn(zr, sldjr, labr, means)
  const = 0.5 * D * math.log(2.0 * math.pi)
  return parts.sum() / B + const
